# baseline (device time: 1066970 ns/iter reference)
import os

import jax
import jax.numpy as jnp
from jax import lax
from jax.experimental import pallas as pl
from jax.experimental.pallas import tpu as pltpu

K = int(os.environ.get("A2A_K", "8"))
L = int(os.environ.get("A2A_L", "16"))
_SKIP_LOCAL = os.environ.get("A2A_SKIP_LOCAL", "0") == "1"
_SKIP_DIRECT = os.environ.get("A2A_SKIP_DIRECT", "0") == "1"
_SKIP_FWD = os.environ.get("A2A_SKIP_FWD", "0") == "1"


def kernel(x):
    m_per, n_glob = x.shape
    n_out = n_glob // 2
    m_glob = 2 * m_per
    half = m_per // 2
    ch = half // K

    def body(x_ref, out_ref, local_sem, dsend, drecv, fsend, frecv):
        my_x = lax.axis_index("x")
        my_y = lax.axis_index("y")
        nbr_x = (1 - my_x, my_y)
        nbr_y = (my_x, 1 - my_y)

        barrier_sem = pltpu.get_barrier_semaphore()
        for nbr in (nbr_x, nbr_y):
            pl.semaphore_signal(
                barrier_sem, inc=1,
                device_id=nbr, device_id_type=pl.DeviceIdType.MESH,
            )
        pl.semaphore_wait(barrier_sem, 2)

        locals_ = []
        if not _SKIP_LOCAL:
            lch = m_per // L
            for k in range(L):
                c = pltpu.make_async_copy(
                    x_ref.at[pl.ds(k * lch, lch), pl.ds(my_x * n_out, n_out)],
                    out_ref.at[pl.ds(my_x * m_per + k * lch, lch), :],
                    local_sem.at[k],
                )
                c.start()
                locals_.append(c)

        src_row0 = my_y * half
        dst_row0 = my_x * m_per + my_y * half
        fwd_row0 = (1 - my_x) * m_per + my_y * half

        directs = []
        if not _SKIP_DIRECT:
            for k in range(K):
                d = pltpu.make_async_remote_copy(
                    src_ref=x_ref.at[
                        pl.ds(src_row0 + k * ch, ch),
                        pl.ds((1 - my_x) * n_out, n_out),
                    ],
                    dst_ref=out_ref.at[pl.ds(dst_row0 + k * ch, ch), :],
                    send_sem=dsend.at[k],
                    recv_sem=drecv.at[k],
                    device_id=nbr_x,
                    device_id_type=pl.DeviceIdType.MESH,
                )
                d.start()
                directs.append(d)

        fwds = []
        if not _SKIP_FWD:
            for k in range(K):
                if not _SKIP_DIRECT:
                    directs[k].wait_recv()
                f = pltpu.make_async_remote_copy(
                    src_ref=out_ref.at[pl.ds(fwd_row0 + k * ch, ch), :],
                    dst_ref=out_ref.at[pl.ds(fwd_row0 + k * ch, ch), :],
                    send_sem=fsend.at[k],
                    recv_sem=frecv.at[k],
                    device_id=nbr_y,
                    device_id_type=pl.DeviceIdType.MESH,
                )
                f.start()
                fwds.append(f)

        for k in range(K):
            if not _SKIP_DIRECT:
                directs[k].wait_send()
                if _SKIP_FWD:
                    directs[k].wait_recv()
            if not _SKIP_FWD:
                fwds[k].wait_send()
                fwds[k].wait_recv()
        for c in locals_:
            c.wait()

    return pl.pallas_call(
        body,
        out_shape=jax.ShapeDtypeStruct((m_glob, n_out), x.dtype),
        in_specs=[pl.BlockSpec(memory_space=pl.ANY)],
        out_specs=pl.BlockSpec(memory_space=pl.ANY),
        scratch_shapes=[
            pltpu.SemaphoreType.DMA((L,)),
            pltpu.SemaphoreType.DMA((K,)),
            pltpu.SemaphoreType.DMA((K,)),
            pltpu.SemaphoreType.DMA((K,)),
            pltpu.SemaphoreType.DMA((K,)),
        ],
        compiler_params=pltpu.CompilerParams(collective_id=0),
    )(x)


# device time: 254914 ns/iter; 4.1856x vs baseline; 4.1856x over previous
import os

import jax
import jax.numpy as jnp
from jax import lax
from jax.experimental import pallas as pl
from jax.experimental.pallas import tpu as pltpu

K = int(os.environ.get("A2A_K", "8"))
L = int(os.environ.get("A2A_L", "16"))
_SKIP_LOCAL = os.environ.get("A2A_SKIP_LOCAL", "0") == "1"
_SKIP_DIRECT = os.environ.get("A2A_SKIP_DIRECT", "0") == "1"
_SKIP_FWD = os.environ.get("A2A_SKIP_FWD", "0") == "1"


def kernel(x):
    m_per, n_glob = x.shape
    n_out = n_glob // 2
    m_glob = 2 * m_per
    half = m_per // 2
    ch = half // K
    lch = m_per // L

    def body(x_ref, out_ref, vbuf, lin_sem, lout_sem, dsend, drecv, fsend, frecv):
        my_x = lax.axis_index("x")
        my_y = lax.axis_index("y")
        nbr_x = (1 - my_x, my_y)
        nbr_y = (my_x, 1 - my_y)

        barrier_sem = pltpu.get_barrier_semaphore()
        for nbr in (nbr_x, nbr_y):
            pl.semaphore_signal(
                barrier_sem, inc=1,
                device_id=nbr, device_id_type=pl.DeviceIdType.MESH,
            )
        pl.semaphore_wait(barrier_sem, 2)

        lins = []
        if not _SKIP_LOCAL:
            for k in range(L):
                c = pltpu.make_async_copy(
                    x_ref.at[pl.ds(k * lch, lch), pl.ds(my_x * n_out, n_out)],
                    vbuf.at[k],
                    lin_sem.at[k],
                )
                c.start()
                lins.append(c)

        src_row0 = my_y * half
        dst_row0 = my_x * m_per + my_y * half
        fwd_row0 = (1 - my_x) * m_per + my_y * half

        directs = []
        if not _SKIP_DIRECT:
            for k in range(K):
                d = pltpu.make_async_remote_copy(
                    src_ref=x_ref.at[
                        pl.ds(src_row0 + k * ch, ch),
                        pl.ds((1 - my_x) * n_out, n_out),
                    ],
                    dst_ref=out_ref.at[pl.ds(dst_row0 + k * ch, ch), :],
                    send_sem=dsend.at[k],
                    recv_sem=drecv.at[k],
                    device_id=nbr_x,
                    device_id_type=pl.DeviceIdType.MESH,
                )
                d.start()
                directs.append(d)

        louts = []

        def _drain_local(n):
            while len(louts) < min(n, len(lins)):
                k = len(louts)
                lins[k].wait()
                c = pltpu.make_async_copy(
                    vbuf.at[k],
                    out_ref.at[pl.ds(my_x * m_per + k * lch, lch), :],
                    lout_sem.at[k],
                )
                c.start()
                louts.append(c)

        fwds = []
        if not _SKIP_FWD:
            for k in range(K):
                if not _SKIP_DIRECT:
                    directs[k].wait_recv()
                f = pltpu.make_async_remote_copy(
                    src_ref=out_ref.at[pl.ds(fwd_row0 + k * ch, ch), :],
                    dst_ref=out_ref.at[pl.ds(fwd_row0 + k * ch, ch), :],
                    send_sem=fsend.at[k],
                    recv_sem=frecv.at[k],
                    device_id=nbr_y,
                    device_id_type=pl.DeviceIdType.MESH,
                )
                f.start()
                fwds.append(f)
                _drain_local(((k + 1) * L) // K)

        _drain_local(L)

        for k in range(K):
            if not _SKIP_DIRECT:
                directs[k].wait_send()
                if _SKIP_FWD:
                    directs[k].wait_recv()
            if not _SKIP_FWD:
                fwds[k].wait_send()
                fwds[k].wait_recv()
        for c in louts:
            c.wait()

    return pl.pallas_call(
        body,
        out_shape=jax.ShapeDtypeStruct((m_glob, n_out), x.dtype),
        in_specs=[pl.BlockSpec(memory_space=pl.ANY)],
        out_specs=pl.BlockSpec(memory_space=pl.ANY),
        scratch_shapes=[
            pltpu.VMEM((L, m_per // L, n_out), x.dtype),
            pltpu.SemaphoreType.DMA((L,)),
            pltpu.SemaphoreType.DMA((L,)),
            pltpu.SemaphoreType.DMA((K,)),
            pltpu.SemaphoreType.DMA((K,)),
            pltpu.SemaphoreType.DMA((K,)),
            pltpu.SemaphoreType.DMA((K,)),
        ],
        compiler_params=pltpu.CompilerParams(collective_id=0),
    )(x)


# device time: 232528 ns/iter; 4.5886x vs baseline; 1.0963x over previous
import os

import jax
import jax.numpy as jnp
from jax import lax
from jax.experimental import pallas as pl
from jax.experimental.pallas import tpu as pltpu

K = int(os.environ.get("A2A_K", "8"))
L = int(os.environ.get("A2A_L", "16"))
_SKIP_LOCAL = os.environ.get("A2A_SKIP_LOCAL", "0") == "1"
_SKIP_DIRECT = os.environ.get("A2A_SKIP_DIRECT", "0") == "1"
_SKIP_FWD = os.environ.get("A2A_SKIP_FWD", "0") == "1"


def kernel(x):
    m_per, n_glob = x.shape
    n_out = n_glob // 2
    m_glob = 2 * m_per
    half = m_per // 2
    ch = half // K
    lch = m_per // L

    def body(
        x_ref, out_ref, vbuf, sbuf,
        lin_sem, lout_sem, sin_sem, dsend, drecv, fsend, frecv,
    ):
        my_x = lax.axis_index("x")
        my_y = lax.axis_index("y")
        nbr_x = (1 - my_x, my_y)
        nbr_y = (my_x, 1 - my_y)

        barrier_sem = pltpu.get_barrier_semaphore()
        for nbr in (nbr_x, nbr_y):
            pl.semaphore_signal(
                barrier_sem, inc=1,
                device_id=nbr, device_id_type=pl.DeviceIdType.MESH,
            )
        pl.semaphore_wait(barrier_sem, 2)

        lins = []
        if not _SKIP_LOCAL:
            for k in range(L):
                c = pltpu.make_async_copy(
                    x_ref.at[pl.ds(k * lch, lch), pl.ds(my_x * n_out, n_out)],
                    vbuf.at[k],
                    lin_sem.at[k],
                )
                c.start()
                lins.append(c)

        src_row0 = my_y * half
        dst_row0 = my_x * m_per + my_y * half
        fwd_row0 = (1 - my_x) * m_per + my_y * half

        directs = []
        if not _SKIP_DIRECT:
            sins = []
            for k in range(K):
                c = pltpu.make_async_copy(
                    x_ref.at[
                        pl.ds(src_row0 + k * ch, ch),
                        pl.ds((1 - my_x) * n_out, n_out),
                    ],
                    sbuf.at[k],
                    sin_sem.at[k],
                )
                c.start()
                sins.append(c)
            for k in range(K):
                sins[k].wait()
                d = pltpu.make_async_remote_copy(
                    src_ref=sbuf.at[k],
                    dst_ref=out_ref.at[pl.ds(dst_row0 + k * ch, ch), :],
                    send_sem=dsend.at[k],
                    recv_sem=drecv.at[k],
                    device_id=nbr_x,
                    device_id_type=pl.DeviceIdType.MESH,
                )
                d.start()
                directs.append(d)

        louts = []

        def _drain_local(n):
            while len(louts) < min(n, len(lins)):
                k = len(louts)
                lins[k].wait()
                c = pltpu.make_async_copy(
                    vbuf.at[k],
                    out_ref.at[pl.ds(my_x * m_per + k * lch, lch), :],
                    lout_sem.at[k],
                )
                c.start()
                louts.append(c)

        fwds = []
        if not _SKIP_FWD:
            for k in range(K):
                if not _SKIP_DIRECT:
                    directs[k].wait_recv()
                f = pltpu.make_async_remote_copy(
                    src_ref=out_ref.at[pl.ds(fwd_row0 + k * ch, ch), :],
                    dst_ref=out_ref.at[pl.ds(fwd_row0 + k * ch, ch), :],
                    send_sem=fsend.at[k],
                    recv_sem=frecv.at[k],
                    device_id=nbr_y,
                    device_id_type=pl.DeviceIdType.MESH,
                )
                f.start()
                fwds.append(f)
                _drain_local(((k + 1) * L) // K)

        _drain_local(L)

        for k in range(K):
            if not _SKIP_DIRECT:
                directs[k].wait_send()
                if _SKIP_FWD:
                    directs[k].wait_recv()
            if not _SKIP_FWD:
                fwds[k].wait_send()
                fwds[k].wait_recv()
        for c in louts:
            c.wait()

    return pl.pallas_call(
        body,
        out_shape=jax.ShapeDtypeStruct((m_glob, n_out), x.dtype),
        in_specs=[pl.BlockSpec(memory_space=pl.ANY)],
        out_specs=pl.BlockSpec(memory_space=pl.ANY),
        scratch_shapes=[
            pltpu.VMEM((L, m_per // L, n_out), x.dtype),
            pltpu.VMEM((K, half // K, n_out), x.dtype),
            pltpu.SemaphoreType.DMA((L,)),
            pltpu.SemaphoreType.DMA((L,)),
            pltpu.SemaphoreType.DMA((K,)),
            pltpu.SemaphoreType.DMA((K,)),
            pltpu.SemaphoreType.DMA((K,)),
            pltpu.SemaphoreType.DMA((K,)),
            pltpu.SemaphoreType.DMA((K,)),
        ],
        compiler_params=pltpu.CompilerParams(
            collective_id=0,
            vmem_limit_bytes=100 * 1024 * 1024,
        ),
    )(x)


# device time: 45786 ns/iter; 23.3034x vs baseline; 5.0786x over previous
import os

import jax
import jax.numpy as jnp
from jax import lax
from jax.experimental import pallas as pl
from jax.experimental.pallas import tpu as pltpu

K = int(os.environ.get("A2A_K", "8"))
L = int(os.environ.get("A2A_L", "16"))
_DST_VMEM = os.environ.get("A2A_DST_VMEM", "0") == "1"
_SKIP_LOCAL = os.environ.get("A2A_SKIP_LOCAL", "0") == "1"
_SKIP_DIRECT = os.environ.get("A2A_SKIP_DIRECT", "0") == "1"
_SKIP_FWD = os.environ.get("A2A_SKIP_FWD", "0") == "1"


def kernel(x):
    m_per, n_glob = x.shape
    n_out = n_glob // 2
    m_glob = 2 * m_per
    half = m_per // 2
    ch = half // K
    lch = m_per // L

    def body(
        x_ref, out_ref, vbuf, sbuf, rbuf,
        lin_sem, lout_sem, sin_sem, dsend, drecv, fsend, frecv,
    ):
        my_x = lax.axis_index("x")
        my_y = lax.axis_index("y")
        nbr_x = (1 - my_x, my_y)
        nbr_y = (my_x, 1 - my_y)

        barrier_sem = pltpu.get_barrier_semaphore()
        for nbr in (nbr_x, nbr_y):
            pl.semaphore_signal(
                barrier_sem, inc=1,
                device_id=nbr, device_id_type=pl.DeviceIdType.MESH,
            )
        pl.semaphore_wait(barrier_sem, 2)

        lins = []
        if not _SKIP_LOCAL:
            for k in range(L):
                c = pltpu.make_async_copy(
                    x_ref.at[pl.ds(k * lch, lch), pl.ds(my_x * n_out, n_out)],
                    vbuf.at[k],
                    lin_sem.at[k],
                )
                c.start()
                lins.append(c)

        src_row0 = my_y * half
        dst_row0 = my_x * m_per + my_y * half
        fwd_row0 = (1 - my_x) * m_per + my_y * half

        directs = []
        if not _SKIP_DIRECT:
            sins = []
            for k in range(K):
                c = pltpu.make_async_copy(
                    x_ref.at[
                        pl.ds(src_row0 + k * ch, ch),
                        pl.ds((1 - my_x) * n_out, n_out),
                    ],
                    sbuf.at[k],
                    sin_sem.at[k],
                )
                c.start()
                sins.append(c)
            for k in range(K):
                sins[k].wait()
                d = pltpu.make_async_remote_copy(
                    src_ref=sbuf.at[k],
                    dst_ref=(
                        rbuf.at[k] if _DST_VMEM
                        else out_ref.at[pl.ds(dst_row0 + k * ch, ch), :]
                    ),
                    send_sem=dsend.at[k],
                    recv_sem=drecv.at[k],
                    device_id=nbr_x,
                    device_id_type=pl.DeviceIdType.MESH,
                )
                d.start()
                directs.append(d)

        louts = []

        def _drain_local(n):
            while len(louts) < min(n, len(lins)):
                k = len(louts)
                lins[k].wait()
                c = pltpu.make_async_copy(
                    vbuf.at[k],
                    out_ref.at[pl.ds(my_x * m_per + k * lch, lch), :],
                    lout_sem.at[k],
                )
                c.start()
                louts.append(c)

        fwds = []
        if not _SKIP_FWD:
            for k in range(K):
                if not _SKIP_DIRECT:
                    directs[k].wait_recv()
                f = pltpu.make_async_remote_copy(
                    src_ref=out_ref.at[pl.ds(fwd_row0 + k * ch, ch), :],
                    dst_ref=out_ref.at[pl.ds(fwd_row0 + k * ch, ch), :],
                    send_sem=fsend.at[k],
                    recv_sem=frecv.at[k],
                    device_id=nbr_y,
                    device_id_type=pl.DeviceIdType.MESH,
                )
                f.start()
                fwds.append(f)
                _drain_local(((k + 1) * L) // K)

        _drain_local(L)

        for k in range(K):
            if not _SKIP_DIRECT:
                directs[k].wait_send()
                if _SKIP_FWD:
                    directs[k].wait_recv()
            if not _SKIP_FWD:
                fwds[k].wait_send()
                fwds[k].wait_recv()
        for c in louts:
            c.wait()

    return pl.pallas_call(
        body,
        out_shape=jax.ShapeDtypeStruct((m_glob, n_out), x.dtype),
        in_specs=[pl.BlockSpec(memory_space=pl.ANY)],
        out_specs=pl.BlockSpec(memory_space=pl.ANY),
        scratch_shapes=[
            pltpu.VMEM((L, m_per // L, n_out), x.dtype),
            pltpu.VMEM((K, half // K, n_out), x.dtype),
            pltpu.VMEM((K, half // K, n_out), x.dtype),
            pltpu.SemaphoreType.DMA((L,)),
            pltpu.SemaphoreType.DMA((L,)),
            pltpu.SemaphoreType.DMA((K,)),
            pltpu.SemaphoreType.DMA((K,)),
            pltpu.SemaphoreType.DMA((K,)),
            pltpu.SemaphoreType.DMA((K,)),
            pltpu.SemaphoreType.DMA((K,)),
        ],
        compiler_params=pltpu.CompilerParams(
            collective_id=0,
            vmem_limit_bytes=100 * 1024 * 1024,
        ),
    )(x)
